# baseline (device time: 252766 ns/iter reference)
import jax
import jax.numpy as jnp
from jax import lax
from jax.experimental import pallas as pl
from jax.experimental.pallas import tpu as pltpu

N_DEV = 4
WIRE_DTYPE = jnp.float8_e4m3fn
N_TILES = 4


def kernel(x, w_mat, scale_x, scale_w):
    m_total, k_per = x.shape
    k_per2, n = w_mat.shape
    assert k_per == k_per2, (x.shape, w_mat.shape)
    m_per = m_total // N_DEV
    n_t = n // N_TILES
    n_h = n_t // 2

    def body(x_ref, w_ref, sx_ref, sw_ref, out_ref,
             x_full, w_all_r, w_all_l, acc_ref,
             send_x, recv_x, send_r, recv_r, send_l, recv_l):
        t = pl.program_id(0)
        my = lax.axis_index("i")
        left = lax.rem(my + N_DEV - 1, N_DEV)
        right = lax.rem(my + 1, N_DEV)
        diag = lax.rem(my + 2, N_DEV)

        def mm(a, b):
            return lax.dot_general(
                a.astype(jnp.bfloat16), b.astype(jnp.bfloat16),
                (((1,), (0,)), ((), ())),
                preferred_element_type=jnp.float32,
            )

        @pl.when(t == 0)
        def _():
            barrier_sem = pltpu.get_barrier_semaphore()
            for nbr in [left, right, diag]:
                pl.semaphore_signal(
                    barrier_sem, inc=1,
                    device_id=(nbr,), device_id_type=pl.DeviceIdType.MESH,
                )
            pl.semaphore_wait(barrier_sem, N_DEV - 1)

            x_full[my] = x_ref[pl.ds(my * m_per, m_per), :]
            for dst in [left, right, diag]:
                pltpu.make_async_remote_copy(
                    src_ref=x_ref.at[pl.ds(dst * m_per, m_per), :],
                    dst_ref=x_full.at[my],
                    send_sem=send_x.at[dst],
                    recv_sem=recv_x.at[my],
                    device_id=(dst,), device_id_type=pl.DeviceIdType.MESH,
                ).start()

        def wait_x_from(srcs):
            for src in srcs:
                pltpu.make_async_remote_copy(
                    src_ref=x_full.at[src], dst_ref=x_full.at[src],
                    send_sem=send_x.at[src], recv_sem=recv_x.at[src],
                    device_id=(src,), device_id_type=pl.DeviceIdType.MESH,
                ).wait_recv()

        in_flight = []
        for h in range(N_DEV - 1):
            o_sr = lax.rem(my + N_DEV - h, N_DEV)
            o_sl = lax.rem(my + h, N_DEV)
            o_rr = lax.rem(my + 2 * N_DEV - h - 1, N_DEV)
            o_rl = lax.rem(my + h + 1, N_DEV)

            src_r = w_ref.at[:, pl.ds(0, n_h)] if h == 0 else w_all_r.at[o_sr]
            src_l = w_ref.at[:, pl.ds(n_h, n_h)] if h == 0 else w_all_l.at[o_sl]
            rd_r = pltpu.make_async_remote_copy(
                src_ref=src_r, dst_ref=w_all_r.at[o_sr],
                send_sem=send_r.at[o_sr], recv_sem=recv_r.at[o_sr],
                device_id=(right,), device_id_type=pl.DeviceIdType.MESH,
            )
            rd_l = pltpu.make_async_remote_copy(
                src_ref=src_l, dst_ref=w_all_l.at[o_sl],
                send_sem=send_l.at[o_sl], recv_sem=recv_l.at[o_sl],
                device_id=(left,), device_id_type=pl.DeviceIdType.MESH,
            )
            rd_r.start()
            rd_l.start()
            in_flight += [rd_r, rd_l]

            o_cr = lax.rem(my + 2 * N_DEV - h, N_DEV)
            o_cl = lax.rem(my + h, N_DEV)
            if h == 0:
                acc_ref[:, pl.ds(0, n_h)] = mm(x_full[my], w_ref[:, pl.ds(0, n_h)])
                acc_ref[:, pl.ds(n_h, n_h)] = mm(x_full[my], w_ref[:, pl.ds(n_h, n_h)])
            else:
                if h == 1:
                    @pl.when(t == 0)
                    def _():
                        wait_x_from([left, right])
                if h == 2:
                    @pl.when(t == 0)
                    def _():
                        wait_x_from([diag])
                acc_ref[:, pl.ds(0, n_h)] = (
                    acc_ref[:, pl.ds(0, n_h)] + mm(x_full[o_cr], w_all_r[o_cr])
                )
                acc_ref[:, pl.ds(n_h, n_h)] = (
                    acc_ref[:, pl.ds(n_h, n_h)] + mm(x_full[o_cl], w_all_l[o_cl])
                )

            pltpu.make_async_remote_copy(
                src_ref=w_all_r.at[o_rr], dst_ref=w_all_r.at[o_rr],
                send_sem=send_r.at[o_rr], recv_sem=recv_r.at[o_rr],
                device_id=(left,), device_id_type=pl.DeviceIdType.MESH,
            ).wait_recv()
            pltpu.make_async_remote_copy(
                src_ref=w_all_l.at[o_rl], dst_ref=w_all_l.at[o_rl],
                send_sem=send_l.at[o_rl], recv_sem=recv_l.at[o_rl],
                device_id=(right,), device_id_type=pl.DeviceIdType.MESH,
            ).wait_recv()

        @pl.when(t == 0)
        def _():
            for dst in [left, right, diag]:
                pltpu.make_async_remote_copy(
                    src_ref=x_full.at[dst], dst_ref=x_full.at[dst],
                    send_sem=send_x.at[dst], recv_sem=recv_x.at[dst],
                    device_id=(dst,), device_id_type=pl.DeviceIdType.MESH,
                ).wait_send()

        o_cr = lax.rem(my + 1, N_DEV)
        o_cl = lax.rem(my + N_DEV - 1, N_DEV)
        scale = sx_ref[0] * sw_ref[0]
        out_ref[:, pl.ds(0, n_h)] = (
            acc_ref[:, pl.ds(0, n_h)] + mm(x_full[o_cr], w_all_r[o_cr])
        ) * scale
        out_ref[:, pl.ds(n_h, n_h)] = (
            acc_ref[:, pl.ds(n_h, n_h)] + mm(x_full[o_cl], w_all_l[o_cl])
        ) * scale

        for rd in in_flight:
            rd.wait_send()

    x8 = x.astype(WIRE_DTYPE)
    w8 = w_mat.astype(WIRE_DTYPE)

    return pl.pallas_call(
        body,
        grid=(N_TILES,),
        out_shape=jax.ShapeDtypeStruct((m_per, n), jnp.float32),
        in_specs=[
            pl.BlockSpec((m_total, k_per), lambda t: (0, 0)),
            pl.BlockSpec((k_per, n_t), lambda t: (0, t)),
            pl.BlockSpec(memory_space=pltpu.SMEM),
            pl.BlockSpec(memory_space=pltpu.SMEM),
        ],
        out_specs=pl.BlockSpec((m_per, n_t), lambda t: (0, t)),
        scratch_shapes=[
            pltpu.VMEM((N_DEV, m_per, k_per), WIRE_DTYPE),
            pltpu.VMEM((N_DEV, k_per, n_h), WIRE_DTYPE),
            pltpu.VMEM((N_DEV, k_per, n_h), WIRE_DTYPE),
            pltpu.VMEM((m_per, n_t), jnp.float32),
            pltpu.SemaphoreType.DMA((N_DEV,)),
            pltpu.SemaphoreType.DMA((N_DEV,)),
            pltpu.SemaphoreType.DMA((N_DEV,)),
            pltpu.SemaphoreType.DMA((N_DEV,)),
            pltpu.SemaphoreType.DMA((N_DEV,)),
            pltpu.SemaphoreType.DMA((N_DEV,)),
        ],
        compiler_params=pltpu.CompilerParams(
            collective_id=0,
            dimension_semantics=("arbitrary",),
            vmem_limit_bytes=63 * 1024 * 1024,
        ),
    )(x8, w8, scale_x, scale_w)


# device time: 244184 ns/iter; 1.0351x vs baseline; 1.0351x over previous
import jax
import jax.numpy as jnp
from jax import lax
from jax.experimental import pallas as pl
from jax.experimental.pallas import tpu as pltpu

N_DEV = 4
WIRE_DTYPE = jnp.float8_e4m3fn
N_TILES = 4


def kernel(x, w_mat, scale_x, scale_w):
    m_total, k_per = x.shape
    k_per2, n = w_mat.shape
    assert k_per == k_per2, (x.shape, w_mat.shape)
    m_per = m_total // N_DEV
    n_t = n // N_TILES
    n_h = n_t // 2

    def body(x_ref, w_ref, sx_ref, sw_ref, out_ref,
             x_full, w_all_r, w_all_l, acc_ref,
             send_x, recv_x, send_r, recv_r, send_l, recv_l):
        t = pl.program_id(0)
        my = lax.axis_index("i")
        left = lax.rem(my + N_DEV - 1, N_DEV)
        right = lax.rem(my + 1, N_DEV)
        diag = lax.rem(my + 2, N_DEV)

        def mm(a, b):
            return lax.dot_general(
                a, b, (((1,), (0,)), ((), ())),
                preferred_element_type=jnp.float32,
            )

        @pl.when(t == 0)
        def _():
            barrier_sem = pltpu.get_barrier_semaphore()
            for nbr in [left, right, diag]:
                pl.semaphore_signal(
                    barrier_sem, inc=1,
                    device_id=(nbr,), device_id_type=pl.DeviceIdType.MESH,
                )
            pl.semaphore_wait(barrier_sem, N_DEV - 1)

            x_full[my] = x_ref[pl.ds(my * m_per, m_per), :]
            for dst in [left, right, diag]:
                pltpu.make_async_remote_copy(
                    src_ref=x_ref.at[pl.ds(dst * m_per, m_per), :],
                    dst_ref=x_full.at[my],
                    send_sem=send_x.at[dst],
                    recv_sem=recv_x.at[my],
                    device_id=(dst,), device_id_type=pl.DeviceIdType.MESH,
                ).start()

        def wait_x_from(srcs):
            for src in srcs:
                pltpu.make_async_remote_copy(
                    src_ref=x_full.at[src], dst_ref=x_full.at[src],
                    send_sem=send_x.at[src], recv_sem=recv_x.at[src],
                    device_id=(src,), device_id_type=pl.DeviceIdType.MESH,
                ).wait_recv()

        in_flight = []
        for h in range(N_DEV - 1):
            o_sr = lax.rem(my + N_DEV - h, N_DEV)
            o_sl = lax.rem(my + h, N_DEV)
            o_rr = lax.rem(my + 2 * N_DEV - h - 1, N_DEV)
            o_rl = lax.rem(my + h + 1, N_DEV)

            src_r = w_ref.at[:, pl.ds(0, n_h)] if h == 0 else w_all_r.at[o_sr]
            src_l = w_ref.at[:, pl.ds(n_h, n_h)] if h == 0 else w_all_l.at[o_sl]
            rd_r = pltpu.make_async_remote_copy(
                src_ref=src_r, dst_ref=w_all_r.at[o_sr],
                send_sem=send_r.at[o_sr], recv_sem=recv_r.at[o_sr],
                device_id=(right,), device_id_type=pl.DeviceIdType.MESH,
            )
            rd_l = pltpu.make_async_remote_copy(
                src_ref=src_l, dst_ref=w_all_l.at[o_sl],
                send_sem=send_l.at[o_sl], recv_sem=recv_l.at[o_sl],
                device_id=(left,), device_id_type=pl.DeviceIdType.MESH,
            )
            rd_r.start()
            rd_l.start()
            in_flight += [rd_r, rd_l]

            o_cr = lax.rem(my + 2 * N_DEV - h, N_DEV)
            o_cl = lax.rem(my + h, N_DEV)
            if h == 0:
                acc_ref[:, pl.ds(0, n_h)] = mm(x_full[my], w_ref[:, pl.ds(0, n_h)])
                acc_ref[:, pl.ds(n_h, n_h)] = mm(x_full[my], w_ref[:, pl.ds(n_h, n_h)])
            else:
                if h == 1:
                    @pl.when(t == 0)
                    def _():
                        wait_x_from([left, right])
                if h == 2:
                    @pl.when(t == 0)
                    def _():
                        wait_x_from([diag])
                acc_ref[:, pl.ds(0, n_h)] = (
                    acc_ref[:, pl.ds(0, n_h)] + mm(x_full[o_cr], w_all_r[o_cr])
                )
                acc_ref[:, pl.ds(n_h, n_h)] = (
                    acc_ref[:, pl.ds(n_h, n_h)] + mm(x_full[o_cl], w_all_l[o_cl])
                )

            pltpu.make_async_remote_copy(
                src_ref=w_all_r.at[o_rr], dst_ref=w_all_r.at[o_rr],
                send_sem=send_r.at[o_rr], recv_sem=recv_r.at[o_rr],
                device_id=(left,), device_id_type=pl.DeviceIdType.MESH,
            ).wait_recv()
            pltpu.make_async_remote_copy(
                src_ref=w_all_l.at[o_rl], dst_ref=w_all_l.at[o_rl],
                send_sem=send_l.at[o_rl], recv_sem=recv_l.at[o_rl],
                device_id=(right,), device_id_type=pl.DeviceIdType.MESH,
            ).wait_recv()

        @pl.when(t == 0)
        def _():
            for dst in [left, right, diag]:
                pltpu.make_async_remote_copy(
                    src_ref=x_full.at[dst], dst_ref=x_full.at[dst],
                    send_sem=send_x.at[dst], recv_sem=recv_x.at[dst],
                    device_id=(dst,), device_id_type=pl.DeviceIdType.MESH,
                ).wait_send()

        o_cr = lax.rem(my + 1, N_DEV)
        o_cl = lax.rem(my + N_DEV - 1, N_DEV)
        scale = sx_ref[0] * sw_ref[0]
        out_ref[:, pl.ds(0, n_h)] = (
            acc_ref[:, pl.ds(0, n_h)] + mm(x_full[o_cr], w_all_r[o_cr])
        ) * scale
        out_ref[:, pl.ds(n_h, n_h)] = (
            acc_ref[:, pl.ds(n_h, n_h)] + mm(x_full[o_cl], w_all_l[o_cl])
        ) * scale

        for rd in in_flight:
            rd.wait_send()

    x8 = x.astype(WIRE_DTYPE)
    w8 = w_mat.astype(WIRE_DTYPE)

    return pl.pallas_call(
        body,
        grid=(N_TILES,),
        out_shape=jax.ShapeDtypeStruct((m_per, n), jnp.float32),
        in_specs=[
            pl.BlockSpec((m_total, k_per), lambda t: (0, 0)),
            pl.BlockSpec((k_per, n_t), lambda t: (0, t)),
            pl.BlockSpec(memory_space=pltpu.SMEM),
            pl.BlockSpec(memory_space=pltpu.SMEM),
        ],
        out_specs=pl.BlockSpec((m_per, n_t), lambda t: (0, t)),
        scratch_shapes=[
            pltpu.VMEM((N_DEV, m_per, k_per), WIRE_DTYPE),
            pltpu.VMEM((N_DEV, k_per, n_h), WIRE_DTYPE),
            pltpu.VMEM((N_DEV, k_per, n_h), WIRE_DTYPE),
            pltpu.VMEM((m_per, n_t), jnp.float32),
            pltpu.SemaphoreType.DMA((N_DEV,)),
            pltpu.SemaphoreType.DMA((N_DEV,)),
            pltpu.SemaphoreType.DMA((N_DEV,)),
            pltpu.SemaphoreType.DMA((N_DEV,)),
            pltpu.SemaphoreType.DMA((N_DEV,)),
            pltpu.SemaphoreType.DMA((N_DEV,)),
        ],
        compiler_params=pltpu.CompilerParams(
            collective_id=0,
            dimension_semantics=("arbitrary",),
            vmem_limit_bytes=63 * 1024 * 1024,
        ),
    )(x8, w8, scale_x, scale_w)


# device time: 236342 ns/iter; 1.0695x vs baseline; 1.0332x over previous
import jax
import jax.numpy as jnp
from jax import lax
from jax.experimental import pallas as pl
from jax.experimental.pallas import tpu as pltpu

N_DEV = 4
WIRE_DTYPE = jnp.float8_e4m3fn
N_TILES = 4


def kernel(x, w_mat, scale_x, scale_w):
    m_total, k_per = x.shape
    k_per2, n = w_mat.shape
    assert k_per == k_per2, (x.shape, w_mat.shape)
    m_per = m_total // N_DEV
    n_t = n // N_TILES
    n_h = n_t // 2

    def body(x_ref, w_ref, sx_ref, sw_ref, out_ref,
             x_full, w_all_r, w_all_l, acc_ref,
             send_x, recv_x, send_r, recv_r, send_l, recv_l):
        t = pl.program_id(0)
        my = lax.axis_index("i")
        left = lax.rem(my + N_DEV - 1, N_DEV)
        right = lax.rem(my + 1, N_DEV)
        diag = lax.rem(my + 2, N_DEV)

        def mm(a, b):
            return lax.dot_general(
                a, b, (((1,), (0,)), ((), ())),
                preferred_element_type=jnp.float32,
            )

        @pl.when(t == 0)
        def _():
            barrier_sem = pltpu.get_barrier_semaphore()
            for nbr in [left, right, diag]:
                pl.semaphore_signal(
                    barrier_sem, inc=1,
                    device_id=(nbr,), device_id_type=pl.DeviceIdType.MESH,
                )
            pl.semaphore_wait(barrier_sem, N_DEV - 1)

            x_full[my] = x_ref[pl.ds(my * m_per, m_per), :]
            for dst in [left, right, diag]:
                pltpu.make_async_remote_copy(
                    src_ref=x_ref.at[pl.ds(dst * m_per, m_per), :],
                    dst_ref=x_full.at[my],
                    send_sem=send_x.at[dst],
                    recv_sem=recv_x.at[my],
                    device_id=(dst,), device_id_type=pl.DeviceIdType.MESH,
                ).start()

        def wait_x_from(srcs):
            for src in srcs:
                pltpu.make_async_remote_copy(
                    src_ref=x_full.at[src], dst_ref=x_full.at[src],
                    send_sem=send_x.at[src], recv_sem=recv_x.at[src],
                    device_id=(src,), device_id_type=pl.DeviceIdType.MESH,
                ).wait_recv()

        @pl.when(t < N_TILES)
        def _():
            pltpu.make_async_remote_copy(
                src_ref=w_ref.at[:, pl.ds(0, n_h)], dst_ref=w_all_r.at[my],
                send_sem=send_r.at[my], recv_sem=recv_r.at[my],
                device_id=(right,), device_id_type=pl.DeviceIdType.MESH,
            ).start()
            pltpu.make_async_remote_copy(
                src_ref=w_ref.at[:, pl.ds(n_h, n_h)], dst_ref=w_all_l.at[my],
                send_sem=send_l.at[my], recv_sem=recv_l.at[my],
                device_id=(left,), device_id_type=pl.DeviceIdType.MESH,
            ).start()

        @pl.when(t > 0)
        def _():
            scale = sx_ref[0] * sw_ref[0]
            out_ref[:, pl.ds(0, n_h)] = (
                acc_ref[:, pl.ds(0, n_h)] + mm(x_full[right], w_all_r[right])
            ) * scale
            out_ref[:, pl.ds(n_h, n_h)] = (
                acc_ref[:, pl.ds(n_h, n_h)] + mm(x_full[left], w_all_l[left])
            ) * scale

        @pl.when(t < N_TILES)
        def _():
            acc_ref[:, pl.ds(0, n_h)] = mm(x_full[my], w_ref[:, pl.ds(0, n_h)])
            acc_ref[:, pl.ds(n_h, n_h)] = mm(x_full[my], w_ref[:, pl.ds(n_h, n_h)])

            in_flight = []
            for h in range(N_DEV - 1):
                o_sr = lax.rem(my + N_DEV - h, N_DEV)
                o_sl = lax.rem(my + h, N_DEV)
                o_rr = lax.rem(my + 2 * N_DEV - h - 1, N_DEV)
                o_rl = lax.rem(my + h + 1, N_DEV)

                if h > 0:
                    rd_r = pltpu.make_async_remote_copy(
                        src_ref=w_all_r.at[o_sr], dst_ref=w_all_r.at[o_sr],
                        send_sem=send_r.at[o_sr], recv_sem=recv_r.at[o_sr],
                        device_id=(right,), device_id_type=pl.DeviceIdType.MESH,
                    )
                    rd_l = pltpu.make_async_remote_copy(
                        src_ref=w_all_l.at[o_sl], dst_ref=w_all_l.at[o_sl],
                        send_sem=send_l.at[o_sl], recv_sem=recv_l.at[o_sl],
                        device_id=(left,), device_id_type=pl.DeviceIdType.MESH,
                    )
                    rd_r.start()
                    rd_l.start()
                    in_flight += [rd_r, rd_l]

                    if h == 1:
                        @pl.when(t == 0)
                        def _():
                            wait_x_from([left, right])
                    if h == 2:
                        @pl.when(t == 0)
                        def _():
                            wait_x_from([diag])

                    o_cr = lax.rem(my + 2 * N_DEV - h, N_DEV)
                    o_cl = lax.rem(my + h, N_DEV)
                    acc_ref[:, pl.ds(0, n_h)] = (
                        acc_ref[:, pl.ds(0, n_h)]
                        + mm(x_full[o_cr], w_all_r[o_cr])
                    )
                    acc_ref[:, pl.ds(n_h, n_h)] = (
                        acc_ref[:, pl.ds(n_h, n_h)]
                        + mm(x_full[o_cl], w_all_l[o_cl])
                    )

                pltpu.make_async_remote_copy(
                    src_ref=w_all_r.at[o_rr], dst_ref=w_all_r.at[o_rr],
                    send_sem=send_r.at[o_rr], recv_sem=recv_r.at[o_rr],
                    device_id=(left,), device_id_type=pl.DeviceIdType.MESH,
                ).wait_recv()
                pltpu.make_async_remote_copy(
                    src_ref=w_all_l.at[o_rl], dst_ref=w_all_l.at[o_rl],
                    send_sem=send_l.at[o_rl], recv_sem=recv_l.at[o_rl],
                    device_id=(right,), device_id_type=pl.DeviceIdType.MESH,
                ).wait_recv()

            for rd in in_flight:
                rd.wait_send()
            pltpu.make_async_remote_copy(
                src_ref=w_ref.at[:, pl.ds(0, n_h)], dst_ref=w_all_r.at[my],
                send_sem=send_r.at[my], recv_sem=recv_r.at[my],
                device_id=(right,), device_id_type=pl.DeviceIdType.MESH,
            ).wait_send()
            pltpu.make_async_remote_copy(
                src_ref=w_ref.at[:, pl.ds(n_h, n_h)], dst_ref=w_all_l.at[my],
                send_sem=send_l.at[my], recv_sem=recv_l.at[my],
                device_id=(left,), device_id_type=pl.DeviceIdType.MESH,
            ).wait_send()

            @pl.when(t == 0)
            def _():
                for dst in [left, right, diag]:
                    pltpu.make_async_remote_copy(
                        src_ref=x_full.at[dst], dst_ref=x_full.at[dst],
                        send_sem=send_x.at[dst], recv_sem=recv_x.at[dst],
                        device_id=(dst,), device_id_type=pl.DeviceIdType.MESH,
                    ).wait_send()

    x8 = x.astype(WIRE_DTYPE)
    w8 = w_mat.astype(WIRE_DTYPE)

    last_tile = N_TILES - 1
    return pl.pallas_call(
        body,
        grid=(N_TILES + 1,),
        out_shape=jax.ShapeDtypeStruct((m_per, n), jnp.float32),
        in_specs=[
            pl.BlockSpec((m_total, k_per), lambda t: (0, 0)),
            pl.BlockSpec((k_per, n_t), lambda t: (0, jnp.minimum(t, last_tile))),
            pl.BlockSpec(memory_space=pltpu.SMEM),
            pl.BlockSpec(memory_space=pltpu.SMEM),
        ],
        out_specs=pl.BlockSpec((m_per, n_t), lambda t: (0, jnp.maximum(t - 1, 0))),
        scratch_shapes=[
            pltpu.VMEM((N_DEV, m_per, k_per), WIRE_DTYPE),
            pltpu.VMEM((N_DEV, k_per, n_h), WIRE_DTYPE),
            pltpu.VMEM((N_DEV, k_per, n_h), WIRE_DTYPE),
            pltpu.VMEM((m_per, n_t), jnp.float32),
            pltpu.SemaphoreType.DMA((N_DEV,)),
            pltpu.SemaphoreType.DMA((N_DEV,)),
            pltpu.SemaphoreType.DMA((N_DEV,)),
            pltpu.SemaphoreType.DMA((N_DEV,)),
            pltpu.SemaphoreType.DMA((N_DEV,)),
            pltpu.SemaphoreType.DMA((N_DEV,)),
        ],
        compiler_params=pltpu.CompilerParams(
            collective_id=0,
            dimension_semantics=("arbitrary",),
            vmem_limit_bytes=63 * 1024 * 1024,
        ),
    )(x8, w8, scale_x, scale_w)
